# baseline (speedup 1.0000x reference)
import functools

import jax
import jax.numpy as jnp
from jax import lax
from jax.experimental import pallas as pl
from jax.experimental.pallas import tpu as pltpu
from jax.experimental.pallas import tpu_sc as plsc

_K1 = 513
_N = 32768
_ROWS = 128
_NWORKERS = 32
_RPW = _ROWS // _NWORKERS
_NV = _N // 16


def _iota16():
    return lax.broadcasted_iota(jnp.int32, (16,), 0)


def _select_bucket(hist_ref, rank, nbuckets):
    ngroups = nbuckets // 16
    iota = _iota16()

    def body(gi, carry):
        running, found, bsel, rrem = carry
        g = (ngroups - 1) - gi
        grp = hist_ref[pl.ds(g * 16, 16)]
        rev = lax.rev(grp, (0,))
        c = jnp.cumsum(rev)
        maskv = (c + running) >= rank
        cnt = jnp.sum(maskv.astype(jnp.int32))
        j = 16 - cnt
        above_j = jnp.sum(jnp.where(iota == j, c - rev, 0))
        hit = jnp.logical_and(found == 0, cnt > 0)
        bsel = jnp.where(hit, g * 16 + 15 - j, bsel)
        rrem = jnp.where(hit, rank - running - above_j, rrem)
        found = jnp.where(hit, 1, found)
        running = running + jnp.sum(grp)
        return running, found, bsel, rrem

    init = (jnp.int32(0), jnp.int32(0), jnp.int32(0), jnp.int32(0))
    _, _, bsel, rrem = lax.fori_loop(0, ngroups, body, init)
    return bsel, rrem


def _zero_hist(hist_ref):
    z = jnp.zeros((16,), jnp.int32)
    for g in range(16):
        hist_ref[pl.ds(g * 16, 16)] = z


def _hist_full(row_ref, hist_ref, shift):
    ones = jnp.ones((16,), jnp.int32)

    def body(i, _):
        for u in range(8):
            base = i * 128 + u * 16
            v = row_ref[pl.ds(base, 16)]
            k = lax.bitcast_convert_type(jnp.maximum(v, 0.0), jnp.int32)
            b = lax.shift_right_logical(k, shift)
            plsc.addupdate_scatter(hist_ref, [b], ones)
        return 0

    lax.fori_loop(0, _NV // 8, body, 0)


def _compact_full(row_ref, dst_ref, bsel):
    def body(i, offv):
        for u in range(4):
            base = i * 64 + u * 16
            v = row_ref[pl.ds(base, 16)]
            k = lax.bitcast_convert_type(jnp.maximum(v, 0.0), jnp.int32)
            sel = lax.shift_right_logical(k, 23) == bsel
            si = sel.astype(jnp.int32)
            excl = jnp.cumsum(si) - si
            plsc.store_scatter(dst_ref, [offv + excl], k, mask=sel)
            offv = offv + plsc.all_reduce_population_count(sel)
        return offv

    offv = lax.fori_loop(0, _NV // 4, body, jnp.zeros((16,), jnp.int32))
    return jnp.max(offv)


def _hist_partial(src_ref, hist_ref, n, shift, maskbits):
    ones = jnp.ones((16,), jnp.int32)
    iota = _iota16()
    trips = (n + 15) >> 4

    def body(i, _):
        base = i * 16
        k = src_ref[pl.ds(base, 16)]
        valid = (base + iota) < n
        b = lax.shift_right_logical(k, shift) & maskbits
        plsc.addupdate_scatter(hist_ref, [b], ones, mask=valid)
        return 0

    lax.fori_loop(0, trips, body, 0)


def _compact_partial(src_ref, dst_ref, n, shift, bsel):
    iota = _iota16()
    trips = (n + 15) >> 4

    def body(i, offv):
        base = i * 16
        k = src_ref[pl.ds(base, 16)]
        valid = (base + iota) < n
        sel = jnp.logical_and(
            (lax.shift_right_logical(k, shift) & 0xFF) == bsel, valid)
        si = sel.astype(jnp.int32)
        excl = jnp.cumsum(si) - si
        plsc.store_scatter(dst_ref, [offv + excl], k, mask=sel)
        return offv + plsc.all_reduce_population_count(sel)

    offv = lax.fori_loop(0, trips, body, jnp.zeros((16,), jnp.int32))
    return jnp.max(offv)


def _sc_kernel(x_hbm, out_hbm, row_v, buf_a, buf_b, hist_v):
    wid = lax.axis_index("s") * 2 + lax.axis_index("c")

    def do_row(j, _):
        r = wid * _RPW + j
        pltpu.sync_copy(x_hbm.at[r], row_v)

        _zero_hist(hist_v)
        _hist_full(row_v, hist_v, 23)
        b1, r1 = _select_bucket(hist_v, jnp.int32(_K1), 256)
        n1 = _compact_full(row_v, buf_a, b1)

        _zero_hist(hist_v)
        _hist_partial(buf_a, hist_v, n1, 15, 0xFF)
        b2, r2 = _select_bucket(hist_v, r1, 256)
        n2 = _compact_partial(buf_a, buf_b, n1, 15, b2)

        _zero_hist(hist_v)
        _hist_partial(buf_b, hist_v, n2, 7, 0xFF)
        b3, r3 = _select_bucket(hist_v, r2, 256)
        n3 = _compact_partial(buf_b, buf_a, n2, 7, b3)

        _zero_hist(hist_v)
        _hist_partial(buf_a, hist_v, n3, 0, 0x7F)
        b4, _ = _select_bucket(hist_v, r3, 128)

        thr_bits = (b1 << 23) | (b2 << 15) | (b3 << 7) | b4
        thrv = lax.bitcast_convert_type(jnp.full((16,), thr_bits, jnp.int32), jnp.float32)

        def mask_body(i, _):
            for u in range(8):
                base = i * 128 + u * 16
                v = row_v[pl.ds(base, 16)]
                row_v[pl.ds(base, 16)] = jnp.maximum(v - thrv, 0.0)
            return 0

        lax.fori_loop(0, _NV // 8, mask_body, 0)
        pltpu.sync_copy(row_v, out_hbm.at[r])
        return 0

    lax.fori_loop(0, _RPW, do_row, 0)


def kernel(x):
    mesh = plsc.VectorSubcoreMesh(core_axis_name="c", subcore_axis_name="s")
    f = functools.partial(
        pl.kernel,
        mesh=mesh,
        compiler_params=pltpu.CompilerParams(needs_layout_passes=False),
        out_type=jax.ShapeDtypeStruct((_ROWS, _N), jnp.float32),
        scratch_types=[
            pltpu.VMEM((_N,), jnp.float32),
            pltpu.VMEM((_N,), jnp.int32),
            pltpu.VMEM((_N,), jnp.int32),
            pltpu.VMEM((256,), jnp.int32),
        ],
    )(_sc_kernel)
    return f(x)


# speedup vs baseline: 1.2229x; 1.2229x over previous
import functools

import jax
import jax.numpy as jnp
from jax import lax
from jax.experimental import pallas as pl
from jax.experimental.pallas import tpu as pltpu
from jax.experimental.pallas import tpu_sc as plsc

_K1 = 513
_N = 32768
_ROWS = 128
_NWORKERS = 32
_RPW = _ROWS // _NWORKERS
_NV = _N // 16
_SEG = _N // 16


def _iota16():
    return lax.broadcasted_iota(jnp.int32, (16,), 0)


def _splat(v):
    return jnp.full((16,), v, jnp.int32)


_GDN = lax.GatherDimensionNumbers(
    offset_dims=(), collapsed_slice_dims=(0,), start_index_map=(0,))


def _take16(vals, idx):
    return lax.gather(vals, idx[:, None], _GDN, (1,),
                      mode=lax.GatherScatterMode.PROMISE_IN_BOUNDS)


def _select_bucket(hist_ref, rank_v, nbuckets):
    ngroups = nbuckets // 16
    fifteen = _splat(15)

    def body(gi, carry):
        running, found, bsel, rrem = carry
        g = (ngroups - 1) - gi
        grp = hist_ref[pl.ds(g * 16, 16)]
        rev = lax.rev(grp, (0,))
        c = jnp.cumsum(rev)
        maskv = (c + running) >= rank_v
        cntv = plsc.all_reduce_population_count(maskv)
        jv = jnp.minimum(16 - cntv, fifteen)
        above = _take16(c - rev, jv)
        tot = _take16(c, fifteen)
        hit = jnp.logical_and(found == 0, cntv > 0)
        bsel = jnp.where(hit, g * 16 + 15 - jv, bsel)
        rrem = jnp.where(hit, rank_v - running - above, rrem)
        found = jnp.where(cntv > 0, _splat(1), found)
        running = running + tot
        return running, found, bsel, rrem

    z = jnp.zeros((16,), jnp.int32)
    _, _, bsel, rrem = lax.fori_loop(0, ngroups, body, (z, z, z, z))
    return bsel, rrem


def _zero_hist(hist_ref):
    z = jnp.zeros((16,), jnp.int32)
    for g in range(16):
        hist_ref[pl.ds(g * 16, 16)] = z


def _hist_full(row_ref, hist_ref):
    ones = jnp.ones((16,), jnp.int32)

    def body(i, _):
        for u in range(8):
            base = i * 128 + u * 16
            v = row_ref[pl.ds(base, 16)]
            k = lax.bitcast_convert_type(jnp.maximum(v, 0.0), jnp.int32)
            b = lax.shift_right_logical(k, 23)
            plsc.addupdate_scatter(hist_ref, [b], ones)
        return 0

    lax.fori_loop(0, _NV // 8, body, 0)


def _compact_full(row_ref, dst_ref, bsel_v, lane_base):
    def body(i, cnt):
        for u in range(4):
            base = i * 64 + u * 16
            v = row_ref[pl.ds(base, 16)]
            k = lax.bitcast_convert_type(jnp.maximum(v, 0.0), jnp.int32)
            sel = lax.shift_right_logical(k, 23) == bsel_v
            plsc.store_scatter(dst_ref, [lane_base + cnt], k, mask=sel)
            cnt = cnt + sel.astype(jnp.int32)
        return cnt

    return lax.fori_loop(0, _NV // 4, body, jnp.zeros((16,), jnp.int32))


def _hist_partial(src_ref, hist_ref, cnt_v, lane_base, shift, maskbits):
    ones = jnp.ones((16,), jnp.int32)
    trips = jnp.max(cnt_v)

    def body(i, _):
        k = plsc.load_gather(src_ref, [lane_base + i])
        valid = i < cnt_v
        b = lax.shift_right_logical(k, shift) & maskbits
        plsc.addupdate_scatter(hist_ref, [b], ones, mask=valid)
        return 0

    lax.fori_loop(0, trips, body, 0)


def _compact_partial(src_ref, cnt_in, lane_base, shift, bsel_v):
    trips = jnp.max(cnt_in)

    def body(i, cnt):
        k = plsc.load_gather(src_ref, [lane_base + i])
        valid = i < cnt_in
        sel = jnp.logical_and(
            (lax.shift_right_logical(k, shift) & 0xFF) == bsel_v, valid)
        plsc.store_scatter(src_ref, [lane_base + cnt], k, mask=sel)
        return cnt + sel.astype(jnp.int32)

    return lax.fori_loop(0, trips, body, jnp.zeros((16,), jnp.int32))


def _mask_pass(row_ref, thrv):
    def body(i, _):
        for u in range(8):
            base = i * 128 + u * 16
            v = row_ref[pl.ds(base, 16)]
            row_ref[pl.ds(base, 16)] = jnp.maximum(v - thrv, 0.0)
        return 0

    lax.fori_loop(0, _NV // 8, body, 0)


def _sc_kernel(x_hbm, out_hbm, row0, row1, buf_a, hist_v, si0, si1, so0, so1):
    wid = lax.axis_index("s") * 2 + lax.axis_index("c")
    base_row = wid * _RPW
    rows = [row0, row1]
    sin = [si0, si1]
    sout = [so0, so1]
    lane_base = _iota16() * _SEG

    in_copies = [None, None]
    out_copies = [None, None]
    in_copies[0] = pltpu.async_copy(x_hbm.at[base_row], row0, si0)

    for j in range(_RPW):
        p = j % 2
        rbuf = rows[p]
        in_copies[p].wait()

        _zero_hist(hist_v)
        _hist_full(rbuf, hist_v)
        b1, r1 = _select_bucket(hist_v, _splat(_K1), 256)
        cnt1 = _compact_full(rbuf, buf_a, b1, lane_base)

        if j + 1 < _RPW:
            q = 1 - p
            if out_copies[q] is not None:
                out_copies[q].wait()
            in_copies[q] = pltpu.async_copy(
                x_hbm.at[base_row + j + 1], rows[q], sin[q])

        _zero_hist(hist_v)
        _hist_partial(buf_a, hist_v, cnt1, lane_base, 15, 0xFF)
        b2, r2 = _select_bucket(hist_v, r1, 256)
        cnt2 = _compact_partial(buf_a, cnt1, lane_base, 15, b2)

        _zero_hist(hist_v)
        _hist_partial(buf_a, hist_v, cnt2, lane_base, 7, 0xFF)
        b3, r3 = _select_bucket(hist_v, r2, 256)
        cnt3 = _compact_partial(buf_a, cnt2, lane_base, 7, b3)

        _zero_hist(hist_v)
        _hist_partial(buf_a, hist_v, cnt3, lane_base, 0, 0x7F)
        b4, _ = _select_bucket(hist_v, r3, 128)

        thr_bits = (b1 << 23) | (b2 << 15) | (b3 << 7) | b4
        thrv = lax.bitcast_convert_type(thr_bits, jnp.float32)
        _mask_pass(rbuf, thrv)
        out_copies[p] = pltpu.async_copy(rbuf, out_hbm.at[base_row + j], sout[p])

    out_copies[0].wait()
    out_copies[1].wait()


def kernel(x):
    mesh = plsc.VectorSubcoreMesh(core_axis_name="c", subcore_axis_name="s")
    f = functools.partial(
        pl.kernel,
        mesh=mesh,
        compiler_params=pltpu.CompilerParams(needs_layout_passes=False),
        out_type=jax.ShapeDtypeStruct((_ROWS, _N), jnp.float32),
        scratch_types=[
            pltpu.VMEM((_N,), jnp.float32),
            pltpu.VMEM((_N,), jnp.float32),
            pltpu.VMEM((_N,), jnp.int32),
            pltpu.VMEM((256,), jnp.int32),
            pltpu.SemaphoreType.DMA,
            pltpu.SemaphoreType.DMA,
            pltpu.SemaphoreType.DMA,
            pltpu.SemaphoreType.DMA,
        ],
    )(_sc_kernel)
    return f(x)


# speedup vs baseline: 2.4897x; 2.0359x over previous
import functools

import jax
import jax.numpy as jnp
from jax import lax
from jax.experimental import pallas as pl
from jax.experimental.pallas import tpu as pltpu
from jax.experimental.pallas import tpu_sc as plsc

_K1 = 513
_N = 32768
_ROWS = 128
_NWORKERS = 32
_RPW = _ROWS // _NWORKERS
_NV = _N // 16
_SEG = _N // 16


def _iota16():
    return lax.broadcasted_iota(jnp.int32, (16,), 0)


def _splat(v):
    return jnp.full((16,), v, jnp.int32)


_GDN = lax.GatherDimensionNumbers(
    offset_dims=(), collapsed_slice_dims=(0,), start_index_map=(0,))


def _take16(vals, idx):
    return lax.gather(vals, idx[:, None], _GDN, (1,),
                      mode=lax.GatherScatterMode.PROMISE_IN_BOUNDS)


def _select_bucket(hist_ref, rank_v, nbuckets):
    ngroups = nbuckets // 16
    fifteen = _splat(15)

    def body(gi, carry):
        running, found, bsel, rrem = carry
        g = (ngroups - 1) - gi
        grp = hist_ref[pl.ds(g * 16, 16)]
        rev = lax.rev(grp, (0,))
        c = jnp.cumsum(rev)
        maskv = (c + running) >= rank_v
        cntv = plsc.all_reduce_population_count(maskv)
        jv = jnp.minimum(16 - cntv, fifteen)
        above = _take16(c - rev, jv)
        tot = _take16(c, fifteen)
        hit = jnp.logical_and(found == 0, cntv > 0)
        bsel = jnp.where(hit, g * 16 + 15 - jv, bsel)
        rrem = jnp.where(hit, rank_v - running - above, rrem)
        found = jnp.where(cntv > 0, _splat(1), found)
        running = running + tot
        return running, found, bsel, rrem

    z = jnp.zeros((16,), jnp.int32)
    _, _, bsel, rrem = lax.fori_loop(0, ngroups, body, (z, z, z, z))
    return bsel, rrem


def _zero_hist(hist_ref):
    z = jnp.zeros((16,), jnp.int32)
    for g in range(16):
        hist_ref[pl.ds(g * 16, 16)] = z


def _hist_full(row_ref, hist_ref):
    ones = jnp.ones((16,), jnp.int32)

    @plsc.parallel_loop(0, _NV, 1, unroll=8)
    def _(i):
        v = row_ref[pl.ds(i * 16, 16)]
        k = lax.bitcast_convert_type(jnp.maximum(v, 0.0), jnp.int32)
        b = lax.shift_right_logical(k, 23)
        plsc.addupdate_scatter(hist_ref, [b], ones)


def _compact_full(row_ref, dst_ref, bsel_v, lane_base):
    @plsc.parallel_loop(0, _NV, 1, unroll=8,
                        carry=jnp.zeros((16,), jnp.int32))
    def cnt1(i, cnt):
        v = row_ref[pl.ds(i * 16, 16)]
        k = lax.bitcast_convert_type(jnp.maximum(v, 0.0), jnp.int32)
        sel = lax.shift_right_logical(k, 23) == bsel_v
        plsc.store_scatter(dst_ref, [lane_base + cnt], k, mask=sel)
        return cnt + sel.astype(jnp.int32)

    return cnt1


def _hist_partial(src_ref, hist_ref, cnt_v, lane_base, shift, maskbits):
    ones = jnp.ones((16,), jnp.int32)
    trips = jnp.max(cnt_v)

    def body(i, _):
        k = plsc.load_gather(src_ref, [lane_base + i])
        valid = i < cnt_v
        b = lax.shift_right_logical(k, shift) & maskbits
        plsc.addupdate_scatter(hist_ref, [b], ones, mask=valid)
        return 0

    lax.fori_loop(0, trips, body, 0)


def _compact_partial(src_ref, cnt_in, lane_base, shift, bsel_v):
    trips = jnp.max(cnt_in)

    def body(i, cnt):
        k = plsc.load_gather(src_ref, [lane_base + i])
        valid = i < cnt_in
        sel = jnp.logical_and(
            (lax.shift_right_logical(k, shift) & 0xFF) == bsel_v, valid)
        plsc.store_scatter(src_ref, [lane_base + cnt], k, mask=sel)
        return cnt + sel.astype(jnp.int32)

    return lax.fori_loop(0, trips, body, jnp.zeros((16,), jnp.int32))


def _mask_pass(row_ref, thrv):
    @plsc.parallel_loop(0, _NV, 1, unroll=8)
    def _(i):
        v = row_ref[pl.ds(i * 16, 16)]
        row_ref[pl.ds(i * 16, 16)] = jnp.maximum(v - thrv, 0.0)


def _sc_kernel(x_hbm, out_hbm, row0, row1, buf_a, hist_v, si0, si1, so0, so1):
    wid = lax.axis_index("s") * 2 + lax.axis_index("c")
    base_row = wid * _RPW
    rows = [row0, row1]
    sin = [si0, si1]
    sout = [so0, so1]
    lane_base = _iota16() * _SEG

    in_copies = [None, None]
    out_copies = [None, None]
    in_copies[0] = pltpu.async_copy(x_hbm.at[base_row], row0, si0)

    for j in range(_RPW):
        p = j % 2
        rbuf = rows[p]
        in_copies[p].wait()

        _zero_hist(hist_v)
        _hist_full(rbuf, hist_v)
        b1, r1 = _select_bucket(hist_v, _splat(_K1), 256)
        cnt1 = _compact_full(rbuf, buf_a, b1, lane_base)

        if j + 1 < _RPW:
            q = 1 - p
            if out_copies[q] is not None:
                out_copies[q].wait()
            in_copies[q] = pltpu.async_copy(
                x_hbm.at[base_row + j + 1], rows[q], sin[q])

        _zero_hist(hist_v)
        _hist_partial(buf_a, hist_v, cnt1, lane_base, 15, 0xFF)
        b2, r2 = _select_bucket(hist_v, r1, 256)
        cnt2 = _compact_partial(buf_a, cnt1, lane_base, 15, b2)

        _zero_hist(hist_v)
        _hist_partial(buf_a, hist_v, cnt2, lane_base, 7, 0xFF)
        b3, r3 = _select_bucket(hist_v, r2, 256)
        cnt3 = _compact_partial(buf_a, cnt2, lane_base, 7, b3)

        _zero_hist(hist_v)
        _hist_partial(buf_a, hist_v, cnt3, lane_base, 0, 0x7F)
        b4, _ = _select_bucket(hist_v, r3, 128)

        thr_bits = (b1 << 23) | (b2 << 15) | (b3 << 7) | b4
        thrv = lax.bitcast_convert_type(thr_bits, jnp.float32)
        _mask_pass(rbuf, thrv)
        out_copies[p] = pltpu.async_copy(rbuf, out_hbm.at[base_row + j], sout[p])

    out_copies[0].wait()
    out_copies[1].wait()


def kernel(x):
    mesh = plsc.VectorSubcoreMesh(core_axis_name="c", subcore_axis_name="s")
    f = functools.partial(
        pl.kernel,
        mesh=mesh,
        compiler_params=pltpu.CompilerParams(needs_layout_passes=False),
        out_type=jax.ShapeDtypeStruct((_ROWS, _N), jnp.float32),
        scratch_types=[
            pltpu.VMEM((_N,), jnp.float32),
            pltpu.VMEM((_N,), jnp.float32),
            pltpu.VMEM((_N,), jnp.int32),
            pltpu.VMEM((256,), jnp.int32),
            pltpu.SemaphoreType.DMA,
            pltpu.SemaphoreType.DMA,
            pltpu.SemaphoreType.DMA,
            pltpu.SemaphoreType.DMA,
        ],
    )(_sc_kernel)
    return f(x)


# speedup vs baseline: 3.9736x; 1.5960x over previous
import functools

import jax
import jax.numpy as jnp
from jax import lax
from jax.experimental import pallas as pl
from jax.experimental.pallas import tpu as pltpu
from jax.experimental.pallas import tpu_sc as plsc

_K1 = 513
_N = 32768
_ROWS = 128
_NWORKERS = 32
_RPW = _ROWS // _NWORKERS
_NV = _N // 16
_SEG = _N // 16


def _iota16():
    return lax.broadcasted_iota(jnp.int32, (16,), 0)


def _splat(v):
    return jnp.full((16,), v, jnp.int32)


_GDN = lax.GatherDimensionNumbers(
    offset_dims=(), collapsed_slice_dims=(0,), start_index_map=(0,))


def _take16(vals, idx):
    return lax.gather(vals, idx[:, None], _GDN, (1,),
                      mode=lax.GatherScatterMode.PROMISE_IN_BOUNDS)


def _select_bucket(hist_ref, rank_v, nbuckets):
    ngroups = nbuckets // 16
    fifteen = _splat(15)

    def body(gi, carry):
        running, found, bsel, rrem = carry
        g = (ngroups - 1) - gi
        grp = hist_ref[pl.ds(g * 16, 16)]
        rev = lax.rev(grp, (0,))
        c = jnp.cumsum(rev)
        maskv = (c + running) >= rank_v
        cntv = plsc.all_reduce_population_count(maskv)
        jv = jnp.minimum(16 - cntv, fifteen)
        above = _take16(c - rev, jv)
        tot = _take16(c, fifteen)
        hit = jnp.logical_and(found == 0, cntv > 0)
        bsel = jnp.where(hit, g * 16 + 15 - jv, bsel)
        rrem = jnp.where(hit, rank_v - running - above, rrem)
        found = jnp.where(cntv > 0, _splat(1), found)
        running = running + tot
        return running, found, bsel, rrem

    z = jnp.zeros((16,), jnp.int32)
    _, _, bsel, rrem = lax.fori_loop(0, ngroups, body, (z, z, z, z))
    return bsel, rrem


def _zero_hist(hist_ref):
    z = jnp.zeros((16,), jnp.int32)
    for g in range(16):
        hist_ref[pl.ds(g * 16, 16)] = z


def _hist_full(row_ref, hist2_ref, iota):
    ones = jnp.ones((16,), jnp.int32)

    @plsc.parallel_loop(0, _NV, 1, unroll=8)
    def _(i):
        v = row_ref[pl.ds(i * 16, 16)]
        k = lax.bitcast_convert_type(jnp.maximum(v, 0.0), jnp.int32)
        slot = (lax.shift_right_logical(k, 19) & 0xFF0) | iota
        plsc.addupdate_scatter(hist2_ref, [slot], ones)


def _zero_hist2(hist2_ref):
    z = jnp.zeros((16,), jnp.int32)

    @plsc.parallel_loop(0, 256, 1, unroll=8)
    def _(b):
        hist2_ref[pl.ds(b * 16, 16)] = z


def _merge_hist2(hist2_ref, hist_ref, iota):
    fifteen = _splat(15)

    def body(g, _):
        tot = jnp.zeros((16,), jnp.int32)
        for m in range(16):
            r = hist2_ref[pl.ds(g * 256 + m * 16, 16)]
            c = jnp.cumsum(r)
            tot = jnp.where(iota == m, _take16(c, fifteen), tot)
        hist_ref[pl.ds(g * 16, 16)] = tot
        return 0

    lax.fori_loop(0, 16, body, 0)


def _compact_full(row_ref, dst_ref, bsel_v, iota):
    @plsc.parallel_loop(0, _NV, 1, unroll=8,
                        carry=jnp.zeros((16,), jnp.int32))
    def cnt1(i, cnt):
        v = row_ref[pl.ds(i * 16, 16)]
        k = lax.bitcast_convert_type(jnp.maximum(v, 0.0), jnp.int32)
        sel = lax.shift_right_logical(k, 23) == bsel_v
        plsc.store_scatter(dst_ref, [cnt * 16 + iota], k, mask=sel)
        return cnt + sel.astype(jnp.int32)

    return cnt1


def _hist_partial(src_ref, hist_ref, cnt_v, iota, shift, maskbits):
    ones = jnp.ones((16,), jnp.int32)
    trips = jnp.max(cnt_v)

    def body(i, _):
        k = src_ref[pl.ds(i * 16, 16)]
        valid = i < cnt_v
        b = lax.shift_right_logical(k, shift) & maskbits
        plsc.addupdate_scatter(hist_ref, [b], ones, mask=valid)
        return 0

    lax.fori_loop(0, trips, body, 0)


def _compact_partial(src_ref, cnt_in, iota, shift, bsel_v):
    trips = jnp.max(cnt_in)

    def body(i, cnt):
        k = src_ref[pl.ds(i * 16, 16)]
        valid = i < cnt_in
        sel = jnp.logical_and(
            (lax.shift_right_logical(k, shift) & 0xFF) == bsel_v, valid)
        plsc.store_scatter(src_ref, [cnt * 16 + iota], k, mask=sel)
        return cnt + sel.astype(jnp.int32)

    return lax.fori_loop(0, trips, body, jnp.zeros((16,), jnp.int32))


def _mask_pass(row_ref, thrv):
    @plsc.parallel_loop(0, _NV, 1, unroll=8)
    def _(i):
        v = row_ref[pl.ds(i * 16, 16)]
        row_ref[pl.ds(i * 16, 16)] = jnp.maximum(v - thrv, 0.0)


def _sc_kernel(x_hbm, out_hbm, row0, row1, buf_a, hist2_v, hist_v,
               si0, si1, so0, so1):
    wid = lax.axis_index("s") * 2 + lax.axis_index("c")
    base_row = wid * _RPW
    rows = [row0, row1]
    sin = [si0, si1]
    sout = [so0, so1]
    iota = _iota16()

    in_copies = [None, None]
    out_copies = [None, None]
    in_copies[0] = pltpu.async_copy(x_hbm.at[base_row], row0, si0)

    for j in range(_RPW):
        p = j % 2
        rbuf = rows[p]
        in_copies[p].wait()

        _zero_hist2(hist2_v)
        _hist_full(rbuf, hist2_v, iota)
        _merge_hist2(hist2_v, hist_v, iota)
        b1, r1 = _select_bucket(hist_v, _splat(_K1), 256)
        cnt1 = _compact_full(rbuf, buf_a, b1, iota)

        if j + 1 < _RPW:
            q = 1 - p
            if out_copies[q] is not None:
                out_copies[q].wait()
            in_copies[q] = pltpu.async_copy(
                x_hbm.at[base_row + j + 1], rows[q], sin[q])

        _zero_hist(hist_v)
        _hist_partial(buf_a, hist_v, cnt1, iota, 15, 0xFF)
        b2, r2 = _select_bucket(hist_v, r1, 256)
        cnt2 = _compact_partial(buf_a, cnt1, iota, 15, b2)

        _zero_hist(hist_v)
        _hist_partial(buf_a, hist_v, cnt2, iota, 7, 0xFF)
        b3, r3 = _select_bucket(hist_v, r2, 256)
        cnt3 = _compact_partial(buf_a, cnt2, iota, 7, b3)

        _zero_hist(hist_v)
        _hist_partial(buf_a, hist_v, cnt3, iota, 0, 0x7F)
        b4, _ = _select_bucket(hist_v, r3, 128)

        thr_bits = (b1 << 23) | (b2 << 15) | (b3 << 7) | b4
        thrv = lax.bitcast_convert_type(thr_bits, jnp.float32)
        _mask_pass(rbuf, thrv)
        out_copies[p] = pltpu.async_copy(rbuf, out_hbm.at[base_row + j], sout[p])

    out_copies[0].wait()
    out_copies[1].wait()


def kernel(x):
    mesh = plsc.VectorSubcoreMesh(core_axis_name="c", subcore_axis_name="s")
    f = functools.partial(
        pl.kernel,
        mesh=mesh,
        compiler_params=pltpu.CompilerParams(needs_layout_passes=False),
        out_type=jax.ShapeDtypeStruct((_ROWS, _N), jnp.float32),
        scratch_types=[
            pltpu.VMEM((_N,), jnp.float32),
            pltpu.VMEM((_N,), jnp.float32),
            pltpu.VMEM((_N,), jnp.int32),
            pltpu.VMEM((4096,), jnp.int32),
            pltpu.VMEM((256,), jnp.int32),
            pltpu.SemaphoreType.DMA,
            pltpu.SemaphoreType.DMA,
            pltpu.SemaphoreType.DMA,
            pltpu.SemaphoreType.DMA,
        ],
    )(_sc_kernel)
    return f(x)


# speedup vs baseline: 4.1058x; 1.0333x over previous
import functools

import jax
import jax.numpy as jnp
from jax import lax
from jax.experimental import pallas as pl
from jax.experimental.pallas import tpu as pltpu
from jax.experimental.pallas import tpu_sc as plsc

_K1 = 513
_N = 32768
_ROWS = 128
_NWORKERS = 32
_RPW = _ROWS // _NWORKERS
_NV = _N // 16
_SEG = _N // 16


def _iota16():
    return lax.broadcasted_iota(jnp.int32, (16,), 0)


def _splat(v):
    return jnp.full((16,), v, jnp.int32)


_GDN = lax.GatherDimensionNumbers(
    offset_dims=(), collapsed_slice_dims=(0,), start_index_map=(0,))


def _take16(vals, idx):
    return lax.gather(vals, idx[:, None], _GDN, (1,),
                      mode=lax.GatherScatterMode.PROMISE_IN_BOUNDS)


def _select_bucket(hist_ref, rank_v, nbuckets):
    ngroups = nbuckets // 16
    fifteen = _splat(15)

    def body(gi, carry):
        running, found, bsel, rrem = carry
        g = (ngroups - 1) - gi
        grp = hist_ref[pl.ds(g * 16, 16)]
        rev = lax.rev(grp, (0,))
        c = jnp.cumsum(rev)
        maskv = (c + running) >= rank_v
        cntv = plsc.all_reduce_population_count(maskv)
        jv = jnp.minimum(16 - cntv, fifteen)
        above = _take16(c - rev, jv)
        tot = _take16(c, fifteen)
        hit = jnp.logical_and(found == 0, cntv > 0)
        bsel = jnp.where(hit, g * 16 + 15 - jv, bsel)
        rrem = jnp.where(hit, rank_v - running - above, rrem)
        found = jnp.where(cntv > 0, _splat(1), found)
        running = running + tot
        return running, found, bsel, rrem

    z = jnp.zeros((16,), jnp.int32)
    _, _, bsel, rrem = lax.fori_loop(0, ngroups, body, (z, z, z, z))
    return bsel, rrem


def _zero_hist(hist_ref):
    z = jnp.zeros((16,), jnp.int32)
    for g in range(16):
        hist_ref[pl.ds(g * 16, 16)] = z


def _hist_full(row_ref, hist2_ref, iota):
    ones = jnp.ones((16,), jnp.int32)

    @plsc.parallel_loop(0, _NV, 1, unroll=8)
    def _(i):
        v = row_ref[pl.ds(i * 16, 16)]
        k = lax.bitcast_convert_type(jnp.maximum(v, 0.0), jnp.int32)
        slot = (lax.shift_right_logical(k, 19) & 0xFF0) | iota
        plsc.addupdate_scatter(hist2_ref, [slot], ones)


def _zero_hist2(hist2_ref):
    z = jnp.zeros((16,), jnp.int32)

    @plsc.parallel_loop(0, 256, 1, unroll=8)
    def _(b):
        hist2_ref[pl.ds(b * 16, 16)] = z


def _merge_hist2(hist2_ref, hist_ref, iota):
    fifteen = _splat(15)

    def body(g, _):
        tot = jnp.zeros((16,), jnp.int32)
        for m in range(16):
            r = hist2_ref[pl.ds(g * 256 + m * 16, 16)]
            c = jnp.cumsum(r)
            tot = jnp.where(iota == m, _take16(c, fifteen), tot)
        hist_ref[pl.ds(g * 16, 16)] = tot
        return 0

    lax.fori_loop(0, 16, body, 0)


def _compact_full(row_ref, dst_ref, bsel_v, iota):
    @plsc.parallel_loop(0, _NV, 1, unroll=8, carry=iota)
    def pos1(i, pos):
        v = row_ref[pl.ds(i * 16, 16)]
        k = lax.bitcast_convert_type(jnp.maximum(v, 0.0), jnp.int32)
        sel = lax.shift_right_logical(k, 23) == bsel_v
        plsc.store_scatter(dst_ref, [pos], k, mask=sel)
        return pos + jnp.where(sel, 16, 0)

    return lax.shift_right_logical(pos1 - iota, 4)


def _hist_partial(src_ref, hist_ref, cnt_v, iota, shift, maskbits):
    ones = jnp.ones((16,), jnp.int32)
    trips = jnp.max(cnt_v)

    @plsc.parallel_loop(0, trips, 1, unroll=2)
    def _(i):
        k = src_ref[pl.ds(i * 16, 16)]
        valid = i < cnt_v
        b = lax.shift_right_logical(k, shift) & maskbits
        plsc.addupdate_scatter(hist_ref, [b], ones, mask=valid)


def _compact_partial(src_ref, cnt_in, iota, shift, bsel_v):
    trips = jnp.max(cnt_in)

    @plsc.parallel_loop(0, trips, 1, unroll=2, carry=iota)
    def pos(i, p):
        k = src_ref[pl.ds(i * 16, 16)]
        valid = i < cnt_in
        sel = jnp.logical_and(
            (lax.shift_right_logical(k, shift) & 0xFF) == bsel_v, valid)
        plsc.store_scatter(src_ref, [p], k, mask=sel)
        return p + jnp.where(sel, 16, 0)

    return lax.shift_right_logical(pos - iota, 4)


def _mask_pass(row_ref, thrv):
    @plsc.parallel_loop(0, _NV, 1, unroll=16)
    def _(i):
        v = row_ref[pl.ds(i * 16, 16)]
        row_ref[pl.ds(i * 16, 16)] = jnp.maximum(v - thrv, 0.0)


def _sc_kernel(x_hbm, out_hbm, row0, row1, buf_a, hist2_v, hist_v,
               si0, si1, so0, so1):
    wid = lax.axis_index("s") * 2 + lax.axis_index("c")
    base_row = wid * _RPW
    rows = [row0, row1]
    sin = [si0, si1]
    sout = [so0, so1]
    iota = _iota16()

    in_copies = [None, None]
    out_copies = [None, None]
    in_copies[0] = pltpu.async_copy(x_hbm.at[base_row], row0, si0)

    for j in range(_RPW):
        p = j % 2
        rbuf = rows[p]
        in_copies[p].wait()

        _zero_hist2(hist2_v)
        _hist_full(rbuf, hist2_v, iota)
        _merge_hist2(hist2_v, hist_v, iota)
        b1, r1 = _select_bucket(hist_v, _splat(_K1), 256)
        cnt1 = _compact_full(rbuf, buf_a, b1, iota)

        if j + 1 < _RPW:
            q = 1 - p
            if out_copies[q] is not None:
                out_copies[q].wait()
            in_copies[q] = pltpu.async_copy(
                x_hbm.at[base_row + j + 1], rows[q], sin[q])

        _zero_hist(hist_v)
        _hist_partial(buf_a, hist_v, cnt1, iota, 15, 0xFF)
        b2, r2 = _select_bucket(hist_v, r1, 256)
        cnt2 = _compact_partial(buf_a, cnt1, iota, 15, b2)

        _zero_hist(hist_v)
        _hist_partial(buf_a, hist_v, cnt2, iota, 7, 0xFF)
        b3, r3 = _select_bucket(hist_v, r2, 256)
        cnt3 = _compact_partial(buf_a, cnt2, iota, 7, b3)

        _zero_hist(hist_v)
        _hist_partial(buf_a, hist_v, cnt3, iota, 0, 0x7F)
        b4, _ = _select_bucket(hist_v, r3, 128)

        thr_bits = (b1 << 23) | (b2 << 15) | (b3 << 7) | b4
        thrv = lax.bitcast_convert_type(thr_bits, jnp.float32)
        _mask_pass(rbuf, thrv)
        out_copies[p] = pltpu.async_copy(rbuf, out_hbm.at[base_row + j], sout[p])

    out_copies[0].wait()
    out_copies[1].wait()


def kernel(x):
    mesh = plsc.VectorSubcoreMesh(core_axis_name="c", subcore_axis_name="s")
    f = functools.partial(
        pl.kernel,
        mesh=mesh,
        compiler_params=pltpu.CompilerParams(needs_layout_passes=False),
        out_type=jax.ShapeDtypeStruct((_ROWS, _N), jnp.float32),
        scratch_types=[
            pltpu.VMEM((_N,), jnp.float32),
            pltpu.VMEM((_N,), jnp.float32),
            pltpu.VMEM((_N,), jnp.int32),
            pltpu.VMEM((4096,), jnp.int32),
            pltpu.VMEM((256,), jnp.int32),
            pltpu.SemaphoreType.DMA,
            pltpu.SemaphoreType.DMA,
            pltpu.SemaphoreType.DMA,
            pltpu.SemaphoreType.DMA,
        ],
    )(_sc_kernel)
    return f(x)
